# Initial kernel scaffold; baseline (speedup 1.0000x reference)
import jax
import jax.numpy as jnp
from jax.experimental import pallas as pl


def kernel(x, p):
    raise NotImplementedError("write your pallas kernel here")


# speedup vs baseline: 1.4461x; 1.4461x over previous
import jax
import jax.numpy as jnp
from jax import lax
from jax.experimental import pallas as pl
from jax.experimental.pallas import tpu as pltpu

_EPS = 1e-6


def _pick_pack(nc: int) -> int:
    for g in (128, 64, 32, 16, 8, 4, 2):
        if nc % g == 0:
            return g
    return 1


def _gem_body(p_ref, x_ref, s_ref, o_ref, *, hw: int):
    p = p_ref[0]
    x = x_ref[...]
    xc = jnp.maximum(x, jnp.float32(_EPS))
    xp = jnp.exp(p * jnp.log(xc))
    sums = lax.dot_general(
        xp.astype(jnp.bfloat16), s_ref[...],
        dimension_numbers=(((1,), (0,)), ((), ())),
        preferred_element_type=jnp.float32)
    m = sums * jnp.float32(1.0 / hw)
    o_ref[...] = jnp.exp(jnp.log(m) * (1.0 / p)).astype(o_ref.dtype)


def kernel(x: jax.Array, p: jax.Array) -> jax.Array:
    N, C, H, W = x.shape
    NC, HW = N * C, H * W
    g = _pick_pack(NC)
    R, L = NC // g, g * HW
    x2d = x.reshape(R, L)

    seg = jnp.arange(L, dtype=jnp.int32) // HW
    s_mat = (seg[:, None] == jnp.arange(g, dtype=jnp.int32)[None, :]
             ).astype(jnp.bfloat16)

    tile_r = R
    for cand in (256, 128, 64, 32, 16, 8):
        if R % cand == 0 and R // cand >= 4:
            tile_r = cand
            break

    out2d = pl.pallas_call(
        lambda pr, xr, sr, orr: _gem_body(pr, xr, sr, orr, hw=HW),
        out_shape=jax.ShapeDtypeStruct((R, g), x.dtype),
        grid=(R // tile_r,),
        in_specs=[
            pl.BlockSpec(memory_space=pltpu.MemorySpace.SMEM),
            pl.BlockSpec((tile_r, L), lambda i: (i, 0)),
            pl.BlockSpec((L, g), lambda i: (0, 0)),
        ],
        out_specs=pl.BlockSpec((tile_r, g), lambda i: (i, 0)),
        compiler_params=pltpu.CompilerParams(
            dimension_semantics=("parallel",),
            vmem_limit_bytes=48 * 1024 * 1024,
        ),
    )(p, x2d, s_mat)

    return out2d.reshape(N, C, 1, 1)


# speedup vs baseline: 34.1319x; 23.6031x over previous
import jax
import jax.numpy as jnp
from jax.experimental import pallas as pl
from jax.experimental.pallas import tpu as pltpu

_EPS = 1e-6


def _gem_body(p_ref, x_ref, o_ref, *, hw: int):
    p = p_ref[0]
    x = x_ref[...]
    xc = jnp.maximum(x, jnp.float32(_EPS))
    xp = jnp.exp(p * jnp.log(xc))
    s = jnp.sum(xp, axis=0)
    m = s * jnp.float32(1.0 / hw)
    o_ref[...] = jnp.exp(jnp.log(m) * (1.0 / p)).astype(o_ref.dtype)


def kernel(x: jax.Array, p: jax.Array) -> jax.Array:
    N, C, H, W = x.shape
    HW = H * W
    xt = jnp.transpose(x, (2, 3, 0, 1)).reshape(HW, N, C)

    tile_n = N
    for cand in (16, 8, 4, 2):
        if N % cand == 0 and N // cand >= 4:
            tile_n = cand
            break

    out2d = pl.pallas_call(
        lambda pr, xr, orr: _gem_body(pr, xr, orr, hw=HW),
        out_shape=jax.ShapeDtypeStruct((N, C), x.dtype),
        grid=(N // tile_n,),
        in_specs=[
            pl.BlockSpec(memory_space=pltpu.MemorySpace.SMEM),
            pl.BlockSpec((HW, tile_n, C), lambda i: (0, i, 0)),
        ],
        out_specs=pl.BlockSpec((tile_n, C), lambda i: (i, 0)),
        compiler_params=pltpu.CompilerParams(
            dimension_semantics=("parallel",),
            vmem_limit_bytes=48 * 1024 * 1024,
        ),
    )(p, xt)

    return out2d.reshape(N, C, 1, 1)
